# baseline (speedup 1.0000x reference)
import functools

import jax
import jax.numpy as jnp
from jax import lax
from jax.experimental import pallas as pl
from jax.experimental.pallas import tpu as pltpu
from jax.experimental.pallas import tpu_sc as plsc

_NC = 2
_NS = 16
_NW = _NC * _NS


def _sc_gather(weight, indices):
    k_short = indices.shape[0]
    d = weight.shape[1]
    rows_per_w = k_short // _NW
    chunk = min(rows_per_w, 128)
    mesh = plsc.VectorSubcoreMesh(core_axis_name="c", subcore_axis_name="s")

    @functools.partial(
        pl.kernel,
        mesh=mesh,
        out_type=jax.ShapeDtypeStruct((k_short, d), jnp.float32),
        scratch_types=[
            pltpu.VMEM((rows_per_w,), jnp.int32),
            pltpu.VMEM((chunk, d), jnp.float32),
            pltpu.VMEM((chunk, d), jnp.float32),
            pltpu.SemaphoreType.DMA,
            pltpu.SemaphoreType.DMA,
            pltpu.SemaphoreType.DMA,
            pltpu.SemaphoreType.DMA,
        ],
    )
    def gather_kernel(w_hbm, idx_hbm, w_out, idx_v, rows_v0, rows_v1,
                      gsem0, gsem1, osem0, osem1):
        rows_b = (rows_v0, rows_v1)
        gsem = (gsem0, gsem1)
        osem = (osem0, osem1)
        wid = lax.axis_index("s") * _NC + lax.axis_index("c")
        base = wid * rows_per_w
        pltpu.sync_copy(idx_hbm.at[pl.ds(base, rows_per_w)], idx_v)

        n_chunks = rows_per_w // chunk

        def fire_gather(c, b):
            return pltpu.async_copy(
                w_hbm.at[idx_v.at[pl.ds(c * chunk, chunk)]], rows_b[b],
                gsem[b])

        def fire_out(c, b):
            return pltpu.async_copy(
                rows_b[b], w_out.at[pl.ds(base + c * chunk, chunk)], osem[b])

        pending_g = {0: fire_gather(0, 0)}
        pending_o = {}
        for c in range(n_chunks):
            b = c & 1
            if c + 1 < n_chunks:
                if c >= 1:
                    pending_o.pop(c - 1).wait()
                pending_g[c + 1] = fire_gather(c + 1, 1 - b)
            pending_g.pop(c).wait()
            pending_o[c] = fire_out(c, b)
        for c in sorted(pending_o):
            pending_o.pop(c).wait()

    return gather_kernel(weight, indices)


_VEC = 16
_PACK = 128


def _sc_ab_gather(att_pack, bias_pack, indices, n_att):
    k_short = indices.shape[0]
    rows_per_w = k_short // _NW
    chunk = min(rows_per_w, 128)
    mesh = plsc.VectorSubcoreMesh(core_axis_name="c", subcore_axis_name="s")
    kvec = jax.ShapeDtypeStruct((k_short,), jnp.float32)
    ivec = pltpu.VMEM((rows_per_w,), jnp.int32)
    rowbuf = pltpu.VMEM((chunk, _PACK), jnp.float32)
    selbuf = pltpu.VMEM((chunk,), jnp.float32)

    @functools.partial(
        pl.kernel,
        mesh=mesh,
        compiler_params=pltpu.CompilerParams(needs_layout_passes=False),
        out_type=[kvec] * (n_att + 1),
        scratch_types=(
            [ivec] * (2 * (n_att + 1) + 1)
            + [rowbuf] * (n_att + 1)
            + [selbuf] * (n_att + 1)
            + [pltpu.SemaphoreType.DMA, pltpu.SemaphoreType.DMA]
        ),
    )
    def ab_kernel(att_hbm, bias_hbm, idx_hbm, *rest):
        outs = rest[:n_att + 1]
        scratch = rest[n_att + 1:]
        idx_v = scratch[0]
        row_vs = scratch[1:1 + n_att + 1]
        lane_vs = scratch[1 + n_att + 1:1 + 2 * (n_att + 1)]
        base_s = 1 + 2 * (n_att + 1)
        bufs = scratch[base_s:base_s + n_att + 1]
        sels = scratch[base_s + n_att + 1:base_s + 2 * (n_att + 1)]
        gsem, osem = scratch[-2], scratch[-1]

        wid = lax.axis_index("s") * _NC + lax.axis_index("c")
        base = wid * rows_per_w
        pltpu.sync_copy(idx_hbm.at[pl.ds(base, rows_per_w)], idx_v)

        @pl.loop(0, rows_per_w, step=_VEC)
        def _(o):
            k = idx_v[pl.ds(o, _VEC)]
            for j in range(n_att):
                t = k * 3 + j
                row_vs[j][pl.ds(o, _VEC)] = lax.shift_right_logical(t, 7)
                lane_vs[j][pl.ds(o, _VEC)] = t & 127
            row_vs[n_att][pl.ds(o, _VEC)] = lax.shift_right_logical(k, 7)
            lane_vs[n_att][pl.ds(o, _VEC)] = k & 127

        @pl.loop(0, rows_per_w, step=chunk)
        def _(cc):
            cps = []
            for t in range(n_att + 1):
                src = att_hbm if t < n_att else bias_hbm
                cps.append(pltpu.async_copy(
                    src.at[row_vs[t].at[pl.ds(cc, chunk)]], bufs[t], gsem))
            for cp in cps:
                cp.wait()

            @pl.loop(0, chunk, step=_VEC)
            def _(g):
                rid = lax.iota(jnp.int32, _VEC) + g
                for t in range(n_att + 1):
                    lo = lane_vs[t][pl.ds(cc + g, _VEC)]
                    sels[t][pl.ds(g, _VEC)] = plsc.load_gather(
                        bufs[t], [rid, lo])

            dst = pl.ds(base + cc, chunk)
            ocps = [pltpu.async_copy(sels[t], outs[t].at[dst], osem)
                    for t in range(n_att + 1)]
            for cp in ocps:
                cp.wait()

    return ab_kernel(att_pack, bias_pack, indices)


def _tc_body(n_j, d, x_ref, w_ref, a0_ref, a1_ref, a2_ref, b_ref, o_ref):
    l0, l1, l2 = a0_ref[...], a1_ref[...], a2_ref[...]
    m = jnp.maximum(jnp.maximum(l0, l1), l2)
    e0, e1, e2 = jnp.exp(l0 - m), jnp.exp(l1 - m), jnp.exp(l2 - m)
    inv = 1.0 / (e0 + e1 + e2)
    a = (e0 * inv, e1 * inv, e2 * inv)
    w_bf = w_ref[...].astype(jnp.bfloat16)
    acc = jnp.broadcast_to(b_ref[...], o_ref.shape)
    for j in range(n_j):
        xj = x_ref[:, j * d:(j + 1) * d]
        g = lax.dot_general(xj, w_bf, (((1,), (1,)), ((), ())),
                            preferred_element_type=jnp.float32)
        acc = acc + g * a[j]
    o_ref[...] = acc


def _tc_matmul(x, w_g, a0, a1, a2, b, kblk=2048):
    bsz, three_i = x.shape
    k_short, d = w_g.shape
    n_j = three_i // d
    row_spec = pl.BlockSpec((1, kblk), lambda i: (0, i))

    return pl.pallas_call(
        functools.partial(_tc_body, n_j, d),
        grid=(k_short // kblk,),
        in_specs=[
            pl.BlockSpec((bsz, three_i), lambda i: (0, 0)),
            pl.BlockSpec((kblk, d), lambda i: (i, 0)),
            row_spec, row_spec, row_spec, row_spec,
        ],
        out_specs=pl.BlockSpec((bsz, kblk), lambda i: (0, i)),
        out_shape=jax.ShapeDtypeStruct((bsz, k_short), jnp.float32),
    )(x, w_g, a0, a1, a2, b)


def _pad_to_rows(flat):
    n = flat.shape[0]
    pad = (-n) % _PACK
    if pad:
        flat = jnp.pad(flat, (0, pad))
    return flat.reshape(-1, _PACK)


def kernel(x, indices, weight, bias, attention_weights):
    k_short = indices.shape[0]
    n_att = attention_weights.shape[1]
    assert n_att == 3
    w_g = _sc_gather(weight, indices)
    att_pack = _pad_to_rows(attention_weights.reshape(-1))
    bias_pack = _pad_to_rows(bias)
    idx_dep = indices + (w_g[0, 0] * 0.0).astype(jnp.int32)
    a0, a1, a2, b = _sc_ab_gather(att_pack, bias_pack, idx_dep, n_att)
    return _tc_matmul(x.astype(jnp.bfloat16), w_g,
                      a0.reshape(1, k_short), a1.reshape(1, k_short),
                      a2.reshape(1, k_short), b.reshape(1, k_short))


# speedup vs baseline: 1.4178x; 1.4178x over previous
import functools

import jax
import jax.numpy as jnp
from jax import lax
from jax.experimental import pallas as pl
from jax.experimental.pallas import tpu as pltpu
from jax.experimental.pallas import tpu_sc as plsc

_NC = 2
_NS = 16
_NW = _NC * _NS


def _sc_gather(weight, indices):
    k_short = indices.shape[0]
    d = weight.shape[1]
    rows_per_w = k_short // _NW
    chunk = min(rows_per_w, 128)
    mesh = plsc.VectorSubcoreMesh(core_axis_name="c", subcore_axis_name="s")

    @functools.partial(
        pl.kernel,
        mesh=mesh,
        out_type=jax.ShapeDtypeStruct((k_short, d), jnp.float32),
        scratch_types=[
            pltpu.VMEM((rows_per_w,), jnp.int32),
            pltpu.VMEM((chunk, d), jnp.float32),
            pltpu.VMEM((chunk, d), jnp.float32),
            pltpu.SemaphoreType.DMA,
            pltpu.SemaphoreType.DMA,
            pltpu.SemaphoreType.DMA,
            pltpu.SemaphoreType.DMA,
        ],
    )
    def gather_kernel(w_hbm, idx_hbm, w_out, idx_v, rows_v0, rows_v1,
                      gsem0, gsem1, osem0, osem1):
        rows_b = (rows_v0, rows_v1)
        gsem = (gsem0, gsem1)
        osem = (osem0, osem1)
        wid = lax.axis_index("s") * _NC + lax.axis_index("c")
        base = wid * rows_per_w
        pltpu.sync_copy(idx_hbm.at[pl.ds(base, rows_per_w)], idx_v)

        n_chunks = rows_per_w // chunk

        def fire_gather(c, b):
            return pltpu.async_copy(
                w_hbm.at[idx_v.at[pl.ds(c * chunk, chunk)]], rows_b[b],
                gsem[b])

        def fire_out(c, b):
            return pltpu.async_copy(
                rows_b[b], w_out.at[pl.ds(base + c * chunk, chunk)], osem[b])

        pending_g = {0: fire_gather(0, 0)}
        pending_o = {}
        for c in range(n_chunks):
            b = c & 1
            if c + 1 < n_chunks:
                if c >= 1:
                    pending_o.pop(c - 1).wait()
                pending_g[c + 1] = fire_gather(c + 1, 1 - b)
            pending_g.pop(c).wait()
            pending_o[c] = fire_out(c, b)
        for c in sorted(pending_o):
            pending_o.pop(c).wait()

    return gather_kernel(weight, indices)


def _tc_body(n_j, d, x_ref, w_ref, at_ref, b_ref, o_ref):
    att = at_ref[...]
    m = jnp.max(att, axis=1, keepdims=True)
    e = jnp.exp(att - m)
    a = e / jnp.sum(e, axis=1, keepdims=True)
    w_f = w_ref[...]
    acc = jnp.broadcast_to(b_ref[...], o_ref.shape)
    for j in range(n_j):
        wj = (w_f * a[:, j:j + 1]).astype(jnp.bfloat16)
        xj = x_ref[:, j * d:(j + 1) * d]
        acc = acc + lax.dot_general(xj, wj, (((1,), (1,)), ((), ())),
                                    preferred_element_type=jnp.float32)
    o_ref[...] = acc


def _tc_matmul(x, w_g, att_g, bias_r, kblk=2048):
    bsz, three_i = x.shape
    k_short, d = w_g.shape
    n_j = three_i // d

    return pl.pallas_call(
        functools.partial(_tc_body, n_j, d),
        grid=(k_short // kblk,),
        in_specs=[
            pl.BlockSpec((bsz, three_i), lambda i: (0, 0)),
            pl.BlockSpec((kblk, d), lambda i: (i, 0)),
            pl.BlockSpec((kblk, n_j), lambda i: (i, 0)),
            pl.BlockSpec((1, kblk), lambda i: (0, i)),
        ],
        out_specs=pl.BlockSpec((bsz, kblk), lambda i: (0, i)),
        out_shape=jax.ShapeDtypeStruct((bsz, k_short), jnp.float32),
    )(x, w_g, att_g, bias_r)


def kernel(x, indices, weight, bias, attention_weights):
    k_short = indices.shape[0]
    w_g = _sc_gather(weight, indices)
    idx_dep = indices + (w_g[0, 0] * 0.0).astype(jnp.int32)
    att_g = attention_weights.at[idx_dep].get(
        mode="promise_in_bounds")
    bias_r = bias.at[indices].get(
        mode="promise_in_bounds").reshape(1, k_short)
    return _tc_matmul(x.astype(jnp.bfloat16), w_g, att_g, bias_r)
